# baseline (speedup 1.0000x reference)
import functools

import jax
import jax.numpy as jnp
from jax import lax
from jax.experimental import pallas as pl
from jax.experimental.pallas import tpu as pltpu
from jax.experimental.pallas import tpu_sc as plsc

NUM_EXPERTS = 64
TOP_K = 8
HIDDEN = 4096
BLOCK_M = 1024
ROWS = 16384
NW = 32
RPW = ROWS // NW


def _logits_block(x_ref, w_ref, logits_ref):
    logits_ref[...] = jnp.dot(x_ref[...], w_ref[...],
                              preferred_element_type=jnp.float32)


def _merge16(a, ai, b, bi):
    br = lax.rev(b, (0,))
    bir = lax.rev(bi, (0,))
    take = a >= br
    m = jnp.where(take, a, br)
    mi = jnp.where(take, ai, bir)
    return plsc.sort_key_val(m, mi, descending=True)


def _sc_topk_body(logits_hbm, w_hbm, i_hbm, slab, wout, iout):
    wid = lax.axis_index("s") * 2 + lax.axis_index("c")
    base = wid * RPW
    pltpu.sync_copy(logits_hbm.at[pl.ds(base, RPW)], slab)

    lane = lax.iota(jnp.int32, 16)
    lane_lt8 = lane < TOP_K

    @plsc.parallel_loop(0, RPW, 1, unroll=4)
    def body(r):
        chunks = []
        for e in range(NUM_EXPERTS // 16):
            v = slab[r, pl.ds(e * 16, 16)]
            ii = lane + e * 16
            chunks.append(plsc.sort_key_val(v, ii, descending=True))
        m01 = _merge16(*chunks[0], *chunks[1])
        m23 = _merge16(*chunks[2], *chunks[3])
        t, ti = _merge16(*m01, *m23)

        ex = jnp.exp(t - jnp.max(t))
        ex8 = jnp.where(lane_lt8, ex, 0.0)
        w = ex8 / jnp.sum(ex8)

        row_idx = jnp.full((16,), r, jnp.int32)
        plsc.store_scatter(wout, [row_idx, lane], w, mask=lane_lt8)
        plsc.store_scatter(iout, [row_idx, lane], ti, mask=lane_lt8)

    pltpu.sync_copy(wout, w_hbm.at[pl.ds(base, RPW)])
    pltpu.sync_copy(iout, i_hbm.at[pl.ds(base, RPW)])


_sc_topk = functools.partial(
    pl.kernel,
    mesh=plsc.VectorSubcoreMesh(core_axis_name="c", subcore_axis_name="s"),
    compiler_params=pltpu.CompilerParams(needs_layout_passes=False,
                                         use_tc_tiling_on_sc=False),
    out_type=[
        jax.ShapeDtypeStruct((ROWS, TOP_K), jnp.float32),
        jax.ShapeDtypeStruct((ROWS, TOP_K), jnp.int32),
    ],
    scratch_types=[
        pltpu.VMEM((RPW, NUM_EXPERTS), jnp.float32),
        pltpu.VMEM((RPW, TOP_K), jnp.float32),
        pltpu.VMEM((RPW, TOP_K), jnp.int32),
    ],
)(_sc_topk_body)


@jax.jit
def kernel(hidden_states, weight):
    x = hidden_states.reshape(-1, HIDDEN)
    wt = weight.T
    logits = pl.pallas_call(
        _logits_block,
        grid=(ROWS // BLOCK_M,),
        in_specs=[
            pl.BlockSpec((BLOCK_M, HIDDEN), lambda i: (i, 0)),
            pl.BlockSpec((HIDDEN, NUM_EXPERTS), lambda i: (0, 0)),
        ],
        out_specs=pl.BlockSpec((BLOCK_M, NUM_EXPERTS), lambda i: (i, 0)),
        out_shape=jax.ShapeDtypeStruct((ROWS, NUM_EXPERTS), jnp.float32),
    )(x, wt)
    weights, indices = _sc_topk(logits)
    return logits, weights, indices


# speedup vs baseline: 1.0050x; 1.0050x over previous
import functools

import jax
import jax.numpy as jnp
from jax import lax
from jax.experimental import pallas as pl
from jax.experimental.pallas import tpu as pltpu
from jax.experimental.pallas import tpu_sc as plsc

NUM_EXPERTS = 64
TOP_K = 8
HIDDEN = 4096
BLOCK_M = 1024
ROWS = 16384
NW = 32
RPW = ROWS // NW
SLAB = RPW * NUM_EXPERTS // 128
OUTR = RPW * TOP_K // 128


def _logits_block(x_ref, w_ref, logits_ref):
    logits_ref[...] = jnp.dot(x_ref[...], w_ref[...],
                              preferred_element_type=jnp.float32)


def _merge16(a, ai, b, bi):
    br = lax.rev(b, (0,))
    bir = lax.rev(bi, (0,))
    take = a >= br
    m = jnp.where(take, a, br)
    mi = jnp.where(take, ai, bir)
    return plsc.sort_key_val(m, mi, descending=True)


def _sc_topk_body(logits_hbm, w_hbm, i_hbm, slab, wout, iout):
    wid = lax.axis_index("s") * 2 + lax.axis_index("c")
    pltpu.sync_copy(logits_hbm.at[pl.ds(wid * SLAB, SLAB)], slab)

    lane = lax.iota(jnp.int32, 16)
    lane_lt8 = lane < TOP_K

    @plsc.parallel_loop(0, RPW, 1, unroll=4)
    def body(r):
        r2 = r // 2
        half = (r % 2) * NUM_EXPERTS
        chunks = []
        for e in range(NUM_EXPERTS // 16):
            v = slab[r2, pl.ds(half + e * 16, 16)]
            ii = lane + e * 16
            chunks.append(plsc.sort_key_val(v, ii, descending=True))
        m01 = _merge16(*chunks[0], *chunks[1])
        m23 = _merge16(*chunks[2], *chunks[3])
        t, ti = _merge16(*m01, *m23)

        ex = jnp.exp(t - jnp.max(t))
        ex8 = jnp.where(lane_lt8, ex, 0.0)
        w = ex8 / jnp.sum(ex8)

        row_idx = jnp.full((16,), r // 16, jnp.int32)
        col_idx = (r % 16) * TOP_K + lane
        plsc.store_scatter(wout, [row_idx, col_idx], w, mask=lane_lt8)
        plsc.store_scatter(iout, [row_idx, col_idx], ti, mask=lane_lt8)

    pltpu.sync_copy(wout, w_hbm.at[pl.ds(wid * OUTR, OUTR)])
    pltpu.sync_copy(iout, i_hbm.at[pl.ds(wid * OUTR, OUTR)])


_sc_topk = functools.partial(
    pl.kernel,
    mesh=plsc.VectorSubcoreMesh(core_axis_name="c", subcore_axis_name="s"),
    compiler_params=pltpu.CompilerParams(needs_layout_passes=False,
                                         use_tc_tiling_on_sc=False),
    out_type=[
        jax.ShapeDtypeStruct((NW * OUTR, 128), jnp.float32),
        jax.ShapeDtypeStruct((NW * OUTR, 128), jnp.int32),
    ],
    scratch_types=[
        pltpu.VMEM((SLAB, 128), jnp.float32),
        pltpu.VMEM((OUTR, 128), jnp.float32),
        pltpu.VMEM((OUTR, 128), jnp.int32),
    ],
)(_sc_topk_body)


@jax.jit
def kernel(hidden_states, weight):
    x = hidden_states.reshape(-1, HIDDEN)
    wt = weight.T
    logits = pl.pallas_call(
        _logits_block,
        grid=(ROWS // BLOCK_M,),
        in_specs=[
            pl.BlockSpec((BLOCK_M, HIDDEN), lambda i: (i, 0)),
            pl.BlockSpec((HIDDEN, NUM_EXPERTS), lambda i: (0, 0)),
        ],
        out_specs=pl.BlockSpec((BLOCK_M, NUM_EXPERTS), lambda i: (i, 0)),
        out_shape=jax.ShapeDtypeStruct((ROWS, NUM_EXPERTS), jnp.float32),
    )(x, wt)
    logits_lin = logits.reshape(ROWS // 2, 2 * NUM_EXPERTS)
    w_lin, i_lin = _sc_topk(logits_lin)
    weights = w_lin.reshape(ROWS, TOP_K)
    indices = i_lin.reshape(ROWS, TOP_K)
    return logits, weights, indices
